# baseline (speedup 1.0000x reference)
import dataclasses
import functools

import jax
import jax.numpy as jnp
from jax import lax
from jax.experimental import pallas as pl
from jax.experimental.pallas import tpu as pltpu
from jax.experimental.pallas import tpu_sc as plsc

EPS = 1e-8
LANES = 16
NUM_CORES = 2
NUM_SUBCORES = 16
NW = NUM_CORES * NUM_SUBCORES
CHUNK = 80


def _prep_body(p_ref, c_ref, t_ref, s_ref, b_ref, tc_ref,
               pn_ref, cn_ref, ex_ref):
    scale = s_ref[0, 0]
    p = p_ref[...]
    pn = jnp.maximum(jnp.sqrt(jnp.sum(p * p, axis=1, keepdims=True)), EPS)
    pn_ref[...] = p * (scale / pn)
    c = c_ref[...]
    cn = jnp.maximum(jnp.sqrt(jnp.sum(c * c, axis=1, keepdims=True)), EPS)
    cn_ref[...] = c / cn
    t = t_ref[...]
    boost = jnp.where(t > 0, 1.0 / (t + 1.0), jnp.zeros_like(t))
    ex_ref[...] = tc_ref[0, 0] * boost + b_ref[0, 0]


def _sc_body(epw, nchunk,
             pn_hbm, cn_hbm, src_hbm, dst_hbm, ex_hbm, out_hbm,
             sidx, didx, srows, drows, exv, outv, sem):
    wid = lax.axis_index("c") * NUM_SUBCORES + lax.axis_index("s")
    base = wid * epw
    nga = CHUNK // LANES
    rows = [jnp.arange(LANES, dtype=jnp.int32) + g * LANES for g in range(nga)]

    @pl.loop(0, nchunk)
    def _chunk(i):
        off = base + i * CHUNK
        pltpu.sync_copy(src_hbm.at[pl.ds(off, CHUNK)], sidx)
        pltpu.sync_copy(dst_hbm.at[pl.ds(off, CHUNK)], didx)
        pltpu.sync_copy(ex_hbm.at[pl.ds(off, CHUNK)], exv)
        cp1 = pltpu.async_copy(pn_hbm.at[sidx], srows, sem)
        cp2 = pltpu.async_copy(cn_hbm.at[didx], drows, sem)
        cp1.wait()
        cp2.wait()

        def dbody(d, accs):
            dv = jnp.full((LANES,), d, dtype=jnp.int32)
            new = []
            for g in range(nga):
                s = plsc.load_gather(srows, [rows[g], dv])
                t = plsc.load_gather(drows, [rows[g], dv])
                new.append(accs[g] + s * t)
            return tuple(new)

        accs = lax.fori_loop(
            0, 128, dbody,
            tuple(jnp.zeros((LANES,), jnp.float32) for _ in range(nga)),
            unroll=4)
        for g in range(nga):
            sl = pl.ds(g * LANES, LANES)
            outv[sl] = accs[g] + exv[sl]
        pltpu.sync_copy(outv, out_hbm.at[pl.ds(off, CHUNK)])


def kernel(patient_embeds, condition_embeds, edge_index, tte, scale, bias,
           time_coeff):
    n, d = patient_embeds.shape
    e = edge_index.shape[1]
    assert d == 128 and e % (NW * CHUNK) == 0

    pn, cn, ex = pl.pallas_call(
        _prep_body,
        out_shape=(
            jax.ShapeDtypeStruct((n, d), jnp.float32),
            jax.ShapeDtypeStruct((n, d), jnp.float32),
            jax.ShapeDtypeStruct((e // 128, 128), jnp.float32),
        ),
        in_specs=[
            pl.BlockSpec(memory_space=pltpu.VMEM),
            pl.BlockSpec(memory_space=pltpu.VMEM),
            pl.BlockSpec(memory_space=pltpu.VMEM),
            pl.BlockSpec(memory_space=pltpu.SMEM),
            pl.BlockSpec(memory_space=pltpu.SMEM),
            pl.BlockSpec(memory_space=pltpu.SMEM),
        ],
    )(patient_embeds, condition_embeds,
      tte.reshape(e // 128, 128),
      scale.reshape(1, 1).astype(jnp.float32),
      bias.reshape(1, 1).astype(jnp.float32),
      time_coeff.reshape(1, 1).astype(jnp.float32))
    ex = ex.reshape(e)

    epw = e // NW
    nchunk = epw // CHUNK
    mesh = plsc.VectorSubcoreMesh(core_axis_name="c", subcore_axis_name="s")
    cp = pltpu.CompilerParams()
    if "needs_layout_passes" in pltpu.CompilerParams.__dataclass_fields__:
        cp = dataclasses.replace(cp, needs_layout_passes=False)
    sc = pl.kernel(
        functools.partial(_sc_body, epw, nchunk),
        out_type=jax.ShapeDtypeStruct((e,), jnp.float32),
        mesh=mesh,
        scratch_types=[
            pltpu.VMEM((CHUNK,), jnp.int32),
            pltpu.VMEM((CHUNK,), jnp.int32),
            pltpu.VMEM((CHUNK, 128), jnp.float32),
            pltpu.VMEM((CHUNK, 128), jnp.float32),
            pltpu.VMEM((CHUNK,), jnp.float32),
            pltpu.VMEM((CHUNK,), jnp.float32),
            pltpu.SemaphoreType.DMA,
        ],
        compiler_params=cp,
    )
    return sc(pn, cn, edge_index[0], edge_index[1], ex)


# speedup vs baseline: 1.2362x; 1.2362x over previous
import dataclasses
import functools

import jax
import jax.numpy as jnp
from jax import lax
from jax.experimental import pallas as pl
from jax.experimental.pallas import tpu as pltpu
from jax.experimental.pallas import tpu_sc as plsc

EPS = 1e-8
LANES = 16
NUM_CORES = 2
NUM_SUBCORES = 16
NW = NUM_CORES * NUM_SUBCORES
CHUNK = 80


def _prep_body(p_ref, c_ref, t_ref, s_ref, b_ref, tc_ref,
               pn_ref, cn_ref, ex_ref):
    scale = s_ref[0, 0]
    p = p_ref[...]
    pn = jnp.maximum(jnp.sqrt(jnp.sum(p * p, axis=1, keepdims=True)), EPS)
    pn_ref[...] = p * (scale / pn)
    c = c_ref[...]
    cn = jnp.maximum(jnp.sqrt(jnp.sum(c * c, axis=1, keepdims=True)), EPS)
    cn_ref[...] = c / cn
    t = t_ref[...]
    boost = jnp.where(t > 0, 1.0 / (t + 1.0), jnp.zeros_like(t))
    ex_ref[...] = tc_ref[0, 0] * boost + b_ref[0, 0]


def _sc_body(epw, nchunk,
             pn_hbm, cn_hbm, src_hbm, dst_hbm, ex_hbm, out_hbm,
             sidx, didx, exv, outv,
             srows0, drows0, srows1, drows1, sem0, sem1):
    wid = lax.axis_index("c") * NUM_SUBCORES + lax.axis_index("s")
    base = wid * epw
    nga = CHUNK // LANES
    rows = [jnp.arange(LANES, dtype=jnp.int32) + g * LANES for g in range(nga)]

    pltpu.sync_copy(src_hbm.at[pl.ds(base, epw)], sidx)
    pltpu.sync_copy(dst_hbm.at[pl.ds(base, epw)], didx)
    pltpu.sync_copy(ex_hbm.at[pl.ds(base, epw)], exv)

    def issue(i, srows, drows, sem):
        sl = pl.ds(i * CHUNK, CHUNK)
        pltpu.async_copy(pn_hbm.at[sidx.at[sl]], srows, sem)
        pltpu.async_copy(cn_hbm.at[didx.at[sl]], drows, sem)

    def drain(srows, drows, sem):
        pltpu.make_async_copy(pn_hbm.at[pl.ds(0, CHUNK)], srows, sem).wait()
        pltpu.make_async_copy(cn_hbm.at[pl.ds(0, CHUNK)], drows, sem).wait()

    def compute(i, srows, drows):
        def dbody(d, accs):
            dv = jnp.full((LANES,), d, dtype=jnp.int32)
            new = []
            for g in range(nga):
                s = plsc.load_gather(srows, [rows[g], dv])
                t = plsc.load_gather(drows, [rows[g], dv])
                new.append(accs[g] + s * t)
            return tuple(new)

        accs = lax.fori_loop(
            0, 128, dbody,
            tuple(jnp.zeros((LANES,), jnp.float32) for _ in range(nga)),
            unroll=4)
        off = i * CHUNK
        for g in range(nga):
            sl = pl.ds(off + g * LANES, LANES)
            outv[sl] = accs[g] + exv[sl]

    issue(0, srows0, drows0, sem0)

    @pl.loop(0, nchunk - 1, step=2)
    def _chunk(i):
        issue(i + 1, srows1, drows1, sem1)
        drain(srows0, drows0, sem0)
        compute(i, srows0, drows0)
        issue(i + 2, srows0, drows0, sem0)
        drain(srows1, drows1, sem1)
        compute(i + 1, srows1, drows1)

    drain(srows0, drows0, sem0)
    compute(nchunk - 1, srows0, drows0)
    pltpu.sync_copy(outv, out_hbm.at[pl.ds(base, epw)])


def kernel(patient_embeds, condition_embeds, edge_index, tte, scale, bias,
           time_coeff):
    n, d = patient_embeds.shape
    e = edge_index.shape[1]
    assert d == 128 and e % (NW * CHUNK) == 0

    pn, cn, ex = pl.pallas_call(
        _prep_body,
        out_shape=(
            jax.ShapeDtypeStruct((n, d), jnp.float32),
            jax.ShapeDtypeStruct((n, d), jnp.float32),
            jax.ShapeDtypeStruct((e // 128, 128), jnp.float32),
        ),
        in_specs=[
            pl.BlockSpec(memory_space=pltpu.VMEM),
            pl.BlockSpec(memory_space=pltpu.VMEM),
            pl.BlockSpec(memory_space=pltpu.VMEM),
            pl.BlockSpec(memory_space=pltpu.SMEM),
            pl.BlockSpec(memory_space=pltpu.SMEM),
            pl.BlockSpec(memory_space=pltpu.SMEM),
        ],
    )(patient_embeds, condition_embeds,
      tte.reshape(e // 128, 128),
      scale.reshape(1, 1).astype(jnp.float32),
      bias.reshape(1, 1).astype(jnp.float32),
      time_coeff.reshape(1, 1).astype(jnp.float32))
    ex = ex.reshape(e)

    epw = e // NW
    nchunk = epw // CHUNK
    assert nchunk % 2 == 1
    mesh = plsc.VectorSubcoreMesh(core_axis_name="c", subcore_axis_name="s")
    cp = pltpu.CompilerParams()
    if "needs_layout_passes" in pltpu.CompilerParams.__dataclass_fields__:
        cp = dataclasses.replace(cp, needs_layout_passes=False)
    sc = pl.kernel(
        functools.partial(_sc_body, epw, nchunk),
        out_type=jax.ShapeDtypeStruct((e,), jnp.float32),
        mesh=mesh,
        scratch_types=[
            pltpu.VMEM((epw,), jnp.int32),
            pltpu.VMEM((epw,), jnp.int32),
            pltpu.VMEM((epw,), jnp.float32),
            pltpu.VMEM((epw,), jnp.float32),
            pltpu.VMEM((CHUNK, 128), jnp.float32),
            pltpu.VMEM((CHUNK, 128), jnp.float32),
            pltpu.VMEM((CHUNK, 128), jnp.float32),
            pltpu.VMEM((CHUNK, 128), jnp.float32),
            pltpu.SemaphoreType.DMA,
            pltpu.SemaphoreType.DMA,
        ],
        compiler_params=cp,
    )
    return sc(pn, cn, edge_index[0], edge_index[1], ex)


# speedup vs baseline: 3.3974x; 2.7482x over previous
import dataclasses
import functools

import jax
import jax.numpy as jnp
from jax import lax
from jax.experimental import pallas as pl
from jax.experimental.pallas import tpu as pltpu
from jax.experimental.pallas import tpu_sc as plsc

EPS = 1e-8
NUM_CORES = 2
NUM_SUBCORES = 16
NW = NUM_CORES * NUM_SUBCORES
NPAD = 10240
GC = 80


def _prep_body(p_ref, c_ref, t_ref, src_ref, dst_ref, s_ref, b_ref, tc_ref,
               pn_ref, cn_ref, ex_ref, fx_ref):
    scale = s_ref[0, 0]
    p = p_ref[...]
    pn = jnp.maximum(jnp.sqrt(jnp.sum(p * p, axis=1, keepdims=True)), EPS)
    pn_ref[...] = p * (scale / pn)
    c = c_ref[...]
    cn = jnp.maximum(jnp.sqrt(jnp.sum(c * c, axis=1, keepdims=True)), EPS)
    n = c.shape[0]
    cn_ref[0:n, :] = c / cn
    cn_ref[n:NPAD, :] = jnp.zeros((NPAD - n, c.shape[1]), jnp.float32)
    t = t_ref[...]
    boost = jnp.where(t > 0, 1.0 / (t + 1.0), jnp.zeros_like(t))
    ex_ref[...] = tc_ref[0, 0] * boost + b_ref[0, 0]
    fx_ref[...] = src_ref[...] * NPAD + dst_ref[...]


def _mm_body(pn_ref, cn_ref, o_ref):
    o_ref[...] = lax.dot_general(
        pn_ref[...], cn_ref[...], (((1,), (1,)), ((), ())),
        preferred_element_type=jnp.float32)


def _sc_body(epw, ng, s_hbm, fx_hbm, ex_hbm, out_hbm, fv, ev, vals, sem):
    wid = lax.axis_index("c") * NUM_SUBCORES + lax.axis_index("s")
    base = wid * epw

    pltpu.sync_copy(fx_hbm.at[pl.ds(base, epw)], fv)
    pltpu.sync_copy(ex_hbm.at[pl.ds(base, epw)], ev)

    @pl.loop(0, ng)
    def _issue(k):
        sl = pl.ds(k * GC, GC)
        pltpu.async_copy(s_hbm.at[fv.at[sl]], vals.at[sl], sem)

    @pl.loop(0, ng)
    def _drain(k):
        pltpu.make_async_copy(
            s_hbm.at[pl.ds(0, GC)], vals.at[pl.ds(0, GC)], sem).wait()

    @pl.loop(0, epw // 16)
    def _add(g):
        sl = pl.ds(g * 16, 16)
        vals[sl] = vals[sl] + ev[sl]

    pltpu.sync_copy(vals, out_hbm.at[pl.ds(base, epw)])


def kernel(patient_embeds, condition_embeds, edge_index, tte, scale, bias,
           time_coeff):
    n, d = patient_embeds.shape
    e = edge_index.shape[1]
    assert d == 128 and n == 10000 and e % (NW * GC) == 0

    pn, cnp, ex, fx = pl.pallas_call(
        _prep_body,
        out_shape=(
            jax.ShapeDtypeStruct((n, d), jnp.float32),
            jax.ShapeDtypeStruct((NPAD, d), jnp.float32),
            jax.ShapeDtypeStruct((e // 128, 128), jnp.float32),
            jax.ShapeDtypeStruct((e // 128, 128), jnp.int32),
        ),
        in_specs=[
            pl.BlockSpec(memory_space=pltpu.VMEM),
            pl.BlockSpec(memory_space=pltpu.VMEM),
            pl.BlockSpec(memory_space=pltpu.VMEM),
            pl.BlockSpec(memory_space=pltpu.VMEM),
            pl.BlockSpec(memory_space=pltpu.VMEM),
            pl.BlockSpec(memory_space=pltpu.SMEM),
            pl.BlockSpec(memory_space=pltpu.SMEM),
            pl.BlockSpec(memory_space=pltpu.SMEM),
        ],
    )(patient_embeds, condition_embeds,
      tte.reshape(e // 128, 128),
      edge_index[0].reshape(e // 128, 128),
      edge_index[1].reshape(e // 128, 128),
      scale.reshape(1, 1).astype(jnp.float32),
      bias.reshape(1, 1).astype(jnp.float32),
      time_coeff.reshape(1, 1).astype(jnp.float32))
    ex = ex.reshape(e)
    fx = fx.reshape(e)

    bi, bj = 1000, 1024
    s = pl.pallas_call(
        _mm_body,
        grid=(n // bi, NPAD // bj),
        out_shape=jax.ShapeDtypeStruct((n, NPAD), jnp.float32),
        in_specs=[
            pl.BlockSpec((bi, d), lambda i, j: (i, 0)),
            pl.BlockSpec((bj, d), lambda i, j: (j, 0)),
        ],
        out_specs=pl.BlockSpec((bi, bj), lambda i, j: (i, j)),
    )(pn, cnp)
    s_flat = s.reshape(n * NPAD)

    epw = e // NW
    ng = epw // GC
    mesh = plsc.VectorSubcoreMesh(core_axis_name="c", subcore_axis_name="s")
    cp = pltpu.CompilerParams()
    if "needs_layout_passes" in pltpu.CompilerParams.__dataclass_fields__:
        cp = dataclasses.replace(cp, needs_layout_passes=False)
    sc = pl.kernel(
        functools.partial(_sc_body, epw, ng),
        out_type=jax.ShapeDtypeStruct((e,), jnp.float32),
        mesh=mesh,
        scratch_types=[
            pltpu.VMEM((epw,), jnp.int32),
            pltpu.VMEM((epw,), jnp.float32),
            pltpu.VMEM((epw,), jnp.float32),
            pltpu.SemaphoreType.DMA,
        ],
        compiler_params=cp,
    )
    return sc(s_flat, fx, ex)
